# baseline (speedup 1.0000x reference)
import jax
import jax.numpy as jnp
from jax import lax
from jax.experimental import pallas as pl
from jax.experimental.pallas import tpu as pltpu

N_PIX = 8192
EMB = 1024
B = 4
CHUNK = 512
NCHUNK = B * N_PIX // CHUNK
NPOS = N_PIX // CHUNK
RING = 8
D = 4


def _body(x_hbm, pos_hbm, o_hbm, xbuf, posbuf, in_sem, out_sem, pos_sem):
    def in_cp(c, s):
        return pltpu.make_async_copy(
            x_hbm.at[pl.ds(c * CHUNK, CHUNK), :], xbuf.at[s], in_sem.at[s])

    def out_cp(c, s):
        return pltpu.make_async_copy(
            xbuf.at[s], o_hbm.at[pl.ds(c * CHUNK, CHUNK), :], out_sem.at[s])

    for c0 in range(D):
        in_cp(c0, c0).start()
    for p in range(NPOS):
        pltpu.make_async_copy(
            pos_hbm.at[pl.ds(p * CHUNK, CHUNK), :], posbuf.at[p], pos_sem.at[p]
        ).start()

    def step(c, _):
        s = lax.rem(c, RING)
        pre = c + D
        pre_s = lax.rem(pre, RING)

        @pl.when(c >= RING - D)
        def _():
            out_cp(c - (RING - D), pre_s).wait()

        @pl.when(pre < NCHUNK)
        def _():
            in_cp(pre, pre_s).start()

        @pl.when(c < NPOS)
        def _():
            pltpu.make_async_copy(
                pos_hbm.at[pl.ds(c * CHUNK, CHUNK), :], posbuf.at[c],
                pos_sem.at[c]).wait()

        in_cp(c, s).wait()
        p = lax.rem(c, NPOS)
        xbuf[s] = xbuf[s] + posbuf[p]
        out_cp(c, s).start()
        return ()

    lax.fori_loop(0, NCHUNK, step, ())

    for k in range(RING - D):
        c = NCHUNK - (RING - D) + k
        out_cp(c, c % RING).wait()


def kernel(x, pos_table):
    x2 = x.reshape(B * N_PIX, EMB)
    out = pl.pallas_call(
        _body,
        in_specs=[
            pl.BlockSpec(memory_space=pl.ANY),
            pl.BlockSpec(memory_space=pl.ANY),
        ],
        out_specs=pl.BlockSpec(memory_space=pl.ANY),
        out_shape=jax.ShapeDtypeStruct((B * N_PIX, EMB), jnp.float32),
        scratch_shapes=[
            pltpu.VMEM((RING, CHUNK, EMB), jnp.float32),
            pltpu.VMEM((NPOS, CHUNK, EMB), jnp.float32),
            pltpu.SemaphoreType.DMA((RING,)),
            pltpu.SemaphoreType.DMA((RING,)),
            pltpu.SemaphoreType.DMA((NPOS,)),
        ],
    )(x2, pos_table)
    return out.reshape(B, N_PIX, EMB)


# speedup vs baseline: 1.0028x; 1.0028x over previous
import jax
import jax.numpy as jnp
from jax.experimental import pallas as pl

N_PIX = 8192
EMB = 1024
B = 4
BLK = 2048


def _add_kernel(x_ref, pos_ref, o_ref):
    o_ref[...] = x_ref[...] + pos_ref[...]


def kernel(x, pos_table):
    x2 = x.reshape(B * N_PIX, EMB)
    out = pl.pallas_call(
        _add_kernel,
        grid=(N_PIX // BLK, B),
        in_specs=[
            pl.BlockSpec((BLK, EMB), lambda i, j: (j * (N_PIX // BLK) + i, 0)),
            pl.BlockSpec((BLK, EMB), lambda i, j: (i, 0)),
        ],
        out_specs=pl.BlockSpec((BLK, EMB), lambda i, j: (j * (N_PIX // BLK) + i, 0)),
        out_shape=jax.ShapeDtypeStruct((B * N_PIX, EMB), jnp.float32),
    )(x2, pos_table)
    return out.reshape(B, N_PIX, EMB)
